# Initial kernel scaffold; baseline (speedup 1.0000x reference)
import jax
import jax.numpy as jnp
from jax.experimental import pallas as pl


def kernel(h, coord_diff, edge_index, W1, b1, W2, b2, Wc1, bc1, Wc2):
    raise NotImplementedError("write your pallas kernel here")


# speedup vs baseline: 1.3241x; 1.3241x over previous
import functools

import jax
import jax.numpy as jnp
from jax.experimental import pallas as pl
from jax.experimental.pallas import tpu as pltpu

N = 10000
E = 320000
D = 128
H = 128


def _silu(x):
    return x * jax.nn.sigmoid(x)


def _proj_body(h_ref, w1_ref, b1_ref, ha_ref, hb_ref):
    h = h_ref[...]
    w1a = w1_ref[0:D, :]
    w1b = w1_ref[D:2 * D, :]
    ha_ref[...] = jnp.dot(h, w1a, preferred_element_type=jnp.float32) + b1_ref[...]
    hb_ref[...] = jnp.dot(h, w1b, preferred_element_type=jnp.float32)


def _proj(h, W1, b1):
    BN = 2000
    return pl.pallas_call(
        _proj_body,
        grid=(N // BN,),
        in_specs=[
            pl.BlockSpec((BN, D), lambda i: (i, 0)),
            pl.BlockSpec((2 * D + 1, H), lambda i: (0, 0)),
            pl.BlockSpec((1, H), lambda i: (0, 0)),
        ],
        out_specs=[
            pl.BlockSpec((BN, H), lambda i: (i, 0)),
            pl.BlockSpec((BN, H), lambda i: (i, 0)),
        ],
        out_shape=[
            jax.ShapeDtypeStruct((N, H), jnp.float32),
            jax.ShapeDtypeStruct((N, H), jnp.float32),
        ],
    )(h, W1, b1)


def _mlp_body(g_ref, cd_ref, w1_ref, w2_ref, b2_ref, wc1_ref, bc1_ref,
              wc2t_ref, out_ref):
    cd = cd_ref[...]
    rad = jnp.sum(cd * cd, axis=1, keepdims=True)
    w1r = w1_ref[2 * D:2 * D + 1, :]
    x1 = _silu(g_ref[...] + rad * w1r)
    x2 = _silu(jnp.dot(x1, w2_ref[...], preferred_element_type=jnp.float32)
               + b2_ref[...])
    x3 = _silu(jnp.dot(x2, wc1_ref[...], preferred_element_type=jnp.float32)
               + bc1_ref[...])
    s = jnp.sum(x3 * wc2t_ref[...], axis=1, keepdims=True)
    t = jnp.clip(cd * s, -100.0, 100.0)
    ones = jnp.ones((t.shape[0], 1), jnp.float32)
    out_ref[...] = jnp.concatenate([t, ones], axis=1)


def _edge_mlp(g, coord_diff, W1, W2, b2, Wc1, bc1, Wc2):
    BE = 2000
    wc2t = Wc2.reshape(1, H)
    return pl.pallas_call(
        _mlp_body,
        grid=(E // BE,),
        in_specs=[
            pl.BlockSpec((BE, H), lambda i: (i, 0)),
            pl.BlockSpec((BE, 3), lambda i: (i, 0)),
            pl.BlockSpec((2 * D + 1, H), lambda i: (0, 0)),
            pl.BlockSpec((H, H), lambda i: (0, 0)),
            pl.BlockSpec((1, H), lambda i: (0, 0)),
            pl.BlockSpec((H, H), lambda i: (0, 0)),
            pl.BlockSpec((1, H), lambda i: (0, 0)),
            pl.BlockSpec((1, H), lambda i: (0, 0)),
        ],
        out_specs=pl.BlockSpec((BE, 4), lambda i: (i, 0)),
        out_shape=jax.ShapeDtypeStruct((E, 4), jnp.float32),
    )(g, coord_diff, W1, W2, b2.reshape(1, H), Wc1, bc1.reshape(1, H), wc2t)


def _combine_body(p_ref, out_ref):
    p = p_ref[0] + p_ref[1]
    cnt = jnp.maximum(p[:, 3:4], 1.0)
    out_ref[...] = p[:, 0:3] / cnt


def _combine(partials):
    return pl.pallas_call(
        _combine_body,
        grid=(1,),
        in_specs=[pl.BlockSpec((2, N, 4), lambda i: (0, 0, 0))],
        out_specs=pl.BlockSpec((N, 3), lambda i: (0, 0)),
        out_shape=jax.ShapeDtypeStruct((N, 3), jnp.float32),
    )(partials)


def kernel(h, coord_diff, edge_index, W1, b1, W2, b2, Wc1, bc1, Wc2):
    row = edge_index[0]
    col = edge_index[1]
    hA, hB = _proj(h, W1, b1.reshape(1, H))
    g = jnp.take(hA, row, axis=0) + jnp.take(hB, col, axis=0)
    trans4 = _edge_mlp(g, coord_diff, W1, W2, b2, Wc1, bc1, Wc2)
    seg = jax.ops.segment_sum(trans4, row, num_segments=N)
    partials = jnp.stack([seg, jnp.zeros_like(seg)], axis=0)
    return _combine(partials)


# speedup vs baseline: 2.5809x; 1.9492x over previous
import functools

import jax
import jax.numpy as jnp
from jax import lax
from jax.experimental import pallas as pl
from jax.experimental.pallas import tpu as pltpu
from jax.experimental.pallas import tpu_sc as plsc

N = 10000
E = 320000
D = 128
H = 128

_NC = 2
_NS = 16
_NW = _NC * _NS
_CK = 128
_NCHUNK = E // _CK
_NJ = (_NCHUNK + _NW - 1) // _NW


def _silu(x):
    return x * jax.nn.sigmoid(x)


def _proj_body(h_ref, w1_ref, b1_ref, ha_ref, hb_ref):
    h = h_ref[...]
    w1a = w1_ref[0:D, :]
    w1b = w1_ref[D:2 * D, :]
    ha_ref[...] = jnp.dot(h, w1a, preferred_element_type=jnp.float32) + b1_ref[...]
    hb_ref[...] = jnp.dot(h, w1b, preferred_element_type=jnp.float32)


def _proj(h, W1, b1):
    BN = 2000
    return pl.pallas_call(
        _proj_body,
        grid=(N // BN,),
        in_specs=[
            pl.BlockSpec((BN, D), lambda i: (i, 0)),
            pl.BlockSpec((2 * D + 1, H), lambda i: (0, 0)),
            pl.BlockSpec((1, H), lambda i: (0, 0)),
        ],
        out_specs=[
            pl.BlockSpec((BN, H), lambda i: (i, 0)),
            pl.BlockSpec((BN, H), lambda i: (i, 0)),
        ],
        out_shape=[
            jax.ShapeDtypeStruct((N, H), jnp.float32),
            jax.ShapeDtypeStruct((N, H), jnp.float32),
        ],
    )(h, W1, b1)


def _gather_body(ha_hbm, hb_hbm, row2d_hbm, col2d_hbm, g_hbm,
                 rowv, colv, bufa, bufb, bufo, sema, semb):
    wid = lax.axis_index("s") * _NC + lax.axis_index("c")

    def chunk_step(j, carry):
        chunk = wid + _NW * j

        @pl.when(chunk < _NCHUNK)
        def _():
            pltpu.sync_copy(row2d_hbm.at[pl.ds(chunk, 1)], rowv)
            pltpu.sync_copy(col2d_hbm.at[pl.ds(chunk, 1)], colv)
            cpa = pltpu.async_copy(ha_hbm.at[rowv.at[0]], bufa, sema)
            cpb = pltpu.async_copy(hb_hbm.at[colv.at[0]], bufb, semb)
            cpa.wait()
            cpb.wait()

            def add_row(r, carry2):
                for cc in range(H // 16):
                    sl = pl.ds(cc * 16, 16)
                    bufo[r, sl] = bufa[r, sl] + bufb[r, sl]
                return carry2

            lax.fori_loop(0, _CK, add_row, 0, unroll=False)
            pltpu.sync_copy(bufo, g_hbm.at[pl.ds(chunk * _CK, _CK)])

        return carry

    lax.fori_loop(0, _NJ, chunk_step, 0, unroll=False)


@functools.partial(jax.jit, donate_argnums=())
def _sc_gather(hA, hB, row2d, col2d):
    mesh = plsc.VectorSubcoreMesh(core_axis_name="c", subcore_axis_name="s")
    f = pl.kernel(
        _gather_body,
        mesh=mesh,
        out_type=jax.ShapeDtypeStruct((E, H), jnp.float32),
        scratch_types=[
            pltpu.VMEM((1, _CK), jnp.int32),
            pltpu.VMEM((1, _CK), jnp.int32),
            pltpu.VMEM((_CK, H), jnp.float32),
            pltpu.VMEM((_CK, H), jnp.float32),
            pltpu.VMEM((_CK, H), jnp.float32),
            pltpu.SemaphoreType.DMA,
            pltpu.SemaphoreType.DMA,
        ],
    )
    return f(hA, hB, row2d, col2d)


def _mlp_body(g_ref, cd_ref, w1_ref, w2_ref, b2_ref, wc1_ref, bc1_ref,
              wc2t_ref, out_ref):
    cd = cd_ref[...]
    rad = jnp.sum(cd * cd, axis=1, keepdims=True)
    w1r = w1_ref[2 * D:2 * D + 1, :]
    x1 = _silu(g_ref[...] + rad * w1r)
    x2 = _silu(jnp.dot(x1, w2_ref[...], preferred_element_type=jnp.float32)
               + b2_ref[...])
    x3 = _silu(jnp.dot(x2, wc1_ref[...], preferred_element_type=jnp.float32)
               + bc1_ref[...])
    s = jnp.sum(x3 * wc2t_ref[...], axis=1, keepdims=True)
    t = jnp.clip(cd * s, -100.0, 100.0)
    ones = jnp.ones((t.shape[0], 1), jnp.float32)
    out_ref[...] = jnp.concatenate([t, ones], axis=1)


def _edge_mlp(g, coord_diff, W1, W2, b2, Wc1, bc1, Wc2):
    BE = 2000
    wc2t = Wc2.reshape(1, H)
    return pl.pallas_call(
        _mlp_body,
        grid=(E // BE,),
        in_specs=[
            pl.BlockSpec((BE, H), lambda i: (i, 0)),
            pl.BlockSpec((BE, 3), lambda i: (i, 0)),
            pl.BlockSpec((2 * D + 1, H), lambda i: (0, 0)),
            pl.BlockSpec((H, H), lambda i: (0, 0)),
            pl.BlockSpec((1, H), lambda i: (0, 0)),
            pl.BlockSpec((H, H), lambda i: (0, 0)),
            pl.BlockSpec((1, H), lambda i: (0, 0)),
            pl.BlockSpec((1, H), lambda i: (0, 0)),
        ],
        out_specs=pl.BlockSpec((BE, 4), lambda i: (i, 0)),
        out_shape=jax.ShapeDtypeStruct((E, 4), jnp.float32),
    )(g, coord_diff, W1, W2, b2.reshape(1, H), Wc1, bc1.reshape(1, H), wc2t)


def _combine_body(p_ref, out_ref):
    p = p_ref[0] + p_ref[1]
    cnt = jnp.maximum(p[:, 3:4], 1.0)
    out_ref[...] = p[:, 0:3] / cnt


def _combine(partials):
    return pl.pallas_call(
        _combine_body,
        grid=(1,),
        in_specs=[pl.BlockSpec((2, N, 4), lambda i: (0, 0, 0))],
        out_specs=pl.BlockSpec((N, 3), lambda i: (0, 0)),
        out_shape=jax.ShapeDtypeStruct((N, 3), jnp.float32),
    )(partials)


def kernel(h, coord_diff, edge_index, W1, b1, W2, b2, Wc1, bc1, Wc2):
    row = edge_index[0]
    col = edge_index[1]
    hA, hB = _proj(h, W1, b1.reshape(1, H))
    row2d = row.reshape(_NCHUNK, _CK)
    col2d = col.reshape(_NCHUNK, _CK)
    g = _sc_gather(hA, hB, row2d, col2d)
    trans4 = _edge_mlp(g, coord_diff, W1, W2, b2, Wc1, bc1, Wc2)
    seg = jax.ops.segment_sum(trans4, row, num_segments=N)
    partials = jnp.stack([seg, jnp.zeros_like(seg)], axis=0)
    return _combine(partials)


# speedup vs baseline: 4.5626x; 1.7678x over previous
import functools

import jax
import jax.numpy as jnp
from jax import lax
from jax.experimental import pallas as pl
from jax.experimental.pallas import tpu as pltpu
from jax.experimental.pallas import tpu_sc as plsc

N = 10000
E = 320000
D = 128
H = 128

_NC = 2
_NS = 16
_NW = _NC * _NS
_CK = 128
_NCHUNK = E // _CK
_NJ = (_NCHUNK + _NW - 1) // _NW


def _silu(x):
    return x * jax.nn.sigmoid(x)


def _proj_body(h_ref, w1_ref, b1_ref, ha_ref, hb_ref):
    h = h_ref[...]
    w1a = w1_ref[0:D, :]
    w1b = w1_ref[D:2 * D, :]
    ha_ref[...] = jnp.dot(h, w1a, preferred_element_type=jnp.float32) + b1_ref[...]
    hb_ref[...] = jnp.dot(h, w1b, preferred_element_type=jnp.float32)


def _proj(h, W1, b1):
    BN = 2000
    return pl.pallas_call(
        _proj_body,
        grid=(N // BN,),
        in_specs=[
            pl.BlockSpec((BN, D), lambda i: (i, 0)),
            pl.BlockSpec((2 * D + 1, H), lambda i: (0, 0)),
            pl.BlockSpec((1, H), lambda i: (0, 0)),
        ],
        out_specs=[
            pl.BlockSpec((BN, H), lambda i: (i, 0)),
            pl.BlockSpec((BN, H), lambda i: (i, 0)),
        ],
        out_shape=[
            jax.ShapeDtypeStruct((N, H), jnp.float32),
            jax.ShapeDtypeStruct((N, H), jnp.float32),
        ],
    )(h, W1, b1)


def _gather_body(ha_hbm, hb_hbm, row2d_hbm, col2d_hbm, g_hbm,
                 rowv, colv, bufa, bufb, bufo, sema, semb):
    wid = lax.axis_index("s") * _NC + lax.axis_index("c")

    def chunk_step(j, carry):
        chunk = wid + _NW * j

        @pl.when(chunk < _NCHUNK)
        def _():
            pltpu.sync_copy(row2d_hbm.at[pl.ds(chunk, 1)], rowv)
            pltpu.sync_copy(col2d_hbm.at[pl.ds(chunk, 1)], colv)
            cpa = pltpu.async_copy(ha_hbm.at[rowv.at[0]], bufa, sema)
            cpb = pltpu.async_copy(hb_hbm.at[colv.at[0]], bufb, semb)
            cpa.wait()
            cpb.wait()

            def add_row(r, carry2):
                for cc in range(H // 16):
                    sl = pl.ds(cc * 16, 16)
                    bufo[r, sl] = bufa[r, sl] + bufb[r, sl]
                return carry2

            lax.fori_loop(0, _CK, add_row, 0, unroll=False)
            pltpu.sync_copy(bufo, g_hbm.at[pl.ds(chunk * _CK, _CK)])

        return carry

    lax.fori_loop(0, _NJ, chunk_step, 0, unroll=False)


@functools.partial(jax.jit, donate_argnums=())
def _sc_gather(hA, hB, row2d, col2d):
    mesh = plsc.VectorSubcoreMesh(core_axis_name="c", subcore_axis_name="s")
    f = pl.kernel(
        _gather_body,
        mesh=mesh,
        out_type=jax.ShapeDtypeStruct((E, H), jnp.float32),
        scratch_types=[
            pltpu.VMEM((1, _CK), jnp.int32),
            pltpu.VMEM((1, _CK), jnp.int32),
            pltpu.VMEM((_CK, H), jnp.float32),
            pltpu.VMEM((_CK, H), jnp.float32),
            pltpu.VMEM((_CK, H), jnp.float32),
            pltpu.SemaphoreType.DMA,
            pltpu.SemaphoreType.DMA,
        ],
    )
    return f(hA, hB, row2d, col2d)


def _mlp_body(g_ref, cd_ref, w1_ref, w2_ref, b2_ref, wc1_ref, bc1_ref,
              wc2t_ref, out_ref):
    cd = cd_ref[...]
    rad = jnp.sum(cd * cd, axis=1, keepdims=True)
    w1r = w1_ref[2 * D:2 * D + 1, :]
    x1 = _silu(g_ref[...] + rad * w1r)
    x2 = _silu(jnp.dot(x1, w2_ref[...], preferred_element_type=jnp.float32)
               + b2_ref[...])
    x3 = _silu(jnp.dot(x2, wc1_ref[...], preferred_element_type=jnp.float32)
               + bc1_ref[...])
    s = jnp.sum(x3 * wc2t_ref[...], axis=1, keepdims=True)
    t = jnp.clip(cd * s, -100.0, 100.0)
    ones = jnp.ones((t.shape[0], 1), jnp.float32)
    t4 = jnp.concatenate([t, ones], axis=1)
    out_ref[...] = t4.T


def _edge_mlp(g, coord_diff, W1, W2, b2, Wc1, bc1, Wc2):
    BE = 2560
    wc2t = Wc2.reshape(1, H)
    return pl.pallas_call(
        _mlp_body,
        grid=(E // BE,),
        in_specs=[
            pl.BlockSpec((BE, H), lambda i: (i, 0)),
            pl.BlockSpec((BE, 3), lambda i: (i, 0)),
            pl.BlockSpec((2 * D + 1, H), lambda i: (0, 0)),
            pl.BlockSpec((H, H), lambda i: (0, 0)),
            pl.BlockSpec((1, H), lambda i: (0, 0)),
            pl.BlockSpec((H, H), lambda i: (0, 0)),
            pl.BlockSpec((1, H), lambda i: (0, 0)),
            pl.BlockSpec((1, H), lambda i: (0, 0)),
        ],
        out_specs=pl.BlockSpec((4, BE), lambda i: (0, i)),
        out_shape=jax.ShapeDtypeStruct((4, E), jnp.float32),
    )(g, coord_diff, W1, W2, b2.reshape(1, H), Wc1, bc1.reshape(1, H), wc2t)


def _scatter_body(trans4t_hbm, row2d_hbm, zeros_hbm, out_hbm,
                  idxv, idx4, tbuf, vbuf, acc):
    cid = lax.axis_index("c")
    sid = lax.axis_index("s")
    wid = sid * _NC + cid

    @pl.when(sid < 10)
    def _():
        pltpu.sync_copy(zeros_hbm.at[pl.ds(sid * 4000, 4000)], vbuf)
        pltpu.sync_copy(vbuf, acc.at[pl.ds(sid * 4000, 4000)])

    plsc.subcore_barrier()

    def chunk_step(j, carry):
        chunk = wid + _NW * j

        @pl.when(chunk < _NCHUNK)
        def _():
            pltpu.sync_copy(row2d_hbm.at[pl.ds(chunk, 1)], idxv)
            pltpu.sync_copy(trans4t_hbm.at[:, pl.ds(chunk * _CK, _CK)], tbuf)
            for gj in range(_CK // 16):
                sl = pl.ds(gj * 16, 16)
                r4 = idxv[0, sl] * 4
                for k in range(4):
                    idx4[k, sl] = r4 + k
            for k in range(4):
                pltpu.sync_copy(tbuf.at[k], acc.at[idx4.at[k]], add=True)

        return carry

    lax.fori_loop(0, _NJ, chunk_step, 0, unroll=False)
    plsc.subcore_barrier()

    @pl.when(sid < 10)
    def _():
        pltpu.sync_copy(acc.at[pl.ds(sid * 4000, 4000)], vbuf)
        pltpu.sync_copy(vbuf, out_hbm.at[pl.ds(cid * 4 * N + sid * 4000, 4000)])


@jax.jit
def _sc_scatter(trans4t, row2d, zeros_flat):
    mesh = plsc.VectorSubcoreMesh(core_axis_name="c", subcore_axis_name="s")
    f = pl.kernel(
        _scatter_body,
        mesh=mesh,
        out_type=jax.ShapeDtypeStruct((2 * 4 * N,), jnp.float32),
        scratch_types=[
            pltpu.VMEM((1, _CK), jnp.int32),
            pltpu.VMEM((4, _CK), jnp.int32),
            pltpu.VMEM((4, _CK), jnp.float32),
            pltpu.VMEM((4000,), jnp.float32),
            pltpu.VMEM_SHARED((4 * N,), jnp.float32),
        ],
    )
    return f(trans4t, row2d, zeros_flat)


def _combine_body(p_ref, out_ref):
    p = p_ref[0] + p_ref[1]
    cnt = jnp.maximum(p[:, 3:4], 1.0)
    out_ref[...] = p[:, 0:3] / cnt


def _combine(partials):
    return pl.pallas_call(
        _combine_body,
        grid=(1,),
        in_specs=[pl.BlockSpec((2, N, 4), lambda i: (0, 0, 0))],
        out_specs=pl.BlockSpec((N, 3), lambda i: (0, 0)),
        out_shape=jax.ShapeDtypeStruct((N, 3), jnp.float32),
    )(partials)


def kernel(h, coord_diff, edge_index, W1, b1, W2, b2, Wc1, bc1, Wc2):
    row = edge_index[0]
    col = edge_index[1]
    hA, hB = _proj(h, W1, b1.reshape(1, H))
    row2d = row.reshape(_NCHUNK, _CK)
    col2d = col.reshape(_NCHUNK, _CK)
    g = _sc_gather(hA, hB, row2d, col2d)
    trans4t = _edge_mlp(g, coord_diff, W1, W2, b2, Wc1, bc1, Wc2)
    zeros_flat = jnp.zeros((4 * N,), jnp.float32)
    partials = _sc_scatter(trans4t, row2d, zeros_flat).reshape(2, N, 4)
    return _combine(partials)
